# baseline (speedup 1.0000x reference)
import functools

import jax
import jax.numpy as jnp
from jax import lax
from jax.experimental import pallas as pl
from jax.experimental.pallas import tpu as pltpu
from jax.experimental.pallas import tpu_sc as plsc

_LANES = 16
_NCHUNK = 4


def _sc_gather(table, idx):
    B = idx.shape[0]
    T = table.shape[0]
    ns = 8
    b_per_w = B // ns
    chunk = b_per_w // _NCHUNK

    mesh = plsc.VectorSubcoreMesh(
        core_axis_name="c", subcore_axis_name="s", num_cores=1, num_subcores=ns
    )

    @functools.partial(
        pl.kernel,
        mesh=mesh,
        out_type=jax.ShapeDtypeStruct((B,), jnp.float32),
        compiler_params=pltpu.CompilerParams(needs_layout_passes=False),
        scratch_types=[
            pltpu.VMEM((T,), jnp.float32),
            pltpu.VMEM((b_per_w,), jnp.int32),
            pltpu.VMEM((b_per_w,), jnp.float32),
            pltpu.SemaphoreType.DMA,
            [pltpu.SemaphoreType.DMA] * _NCHUNK,
            pltpu.SemaphoreType.DMA,
        ],
    )
    def k(table_hbm, idx_hbm, out_hbm, table_v, idx_v, out_v,
          sem_t, sems_i, sem_o):
        wid = lax.axis_index("s")
        base = wid * b_per_w
        cp_t = pltpu.async_copy(table_hbm, table_v, sem_t)
        cp_idx = [
            pltpu.async_copy(
                idx_hbm.at[pl.ds(base + c * chunk, chunk)],
                idx_v.at[pl.ds(c * chunk, chunk)], sems_i[c])
            for c in range(_NCHUNK)
        ]
        cp_t.wait()
        cp_out = []
        for c in range(_NCHUNK):
            cp_idx[c].wait()

            @plsc.parallel_loop(c * chunk, (c + 1) * chunk, step=_LANES,
                                unroll=8)
            def _gather(i):
                ids = idx_v[pl.ds(i, _LANES)]
                out_v[pl.ds(i, _LANES)] = plsc.load_gather(table_v, [ids])

            cp_out.append(pltpu.async_copy(
                out_v.at[pl.ds(c * chunk, chunk)],
                out_hbm.at[pl.ds(base + c * chunk, chunk)], sem_o))
        for cp in cp_out:
            cp.wait()

    return k(table, idx)


def kernel(inData, inIndex, inShape):
    nbatch = inIndex.shape[0]
    out = _sc_gather(inData.astype(jnp.float32), inIndex.astype(jnp.int32))
    return out.reshape((nbatch,) + (1,) * (len(inShape) - 1))


# speedup vs baseline: 1.0170x; 1.0170x over previous
import functools

import jax
import jax.numpy as jnp
from jax import lax
from jax.experimental import pallas as pl
from jax.experimental.pallas import tpu as pltpu
from jax.experimental.pallas import tpu_sc as plsc

_LANES = 16


def _sc_gather(table, idx):
    B = idx.shape[0]
    T = table.shape[0]
    ns = 8
    b_per_w = B // ns

    mesh = plsc.VectorSubcoreMesh(
        core_axis_name="c", subcore_axis_name="s", num_cores=1, num_subcores=ns
    )

    @functools.partial(
        pl.kernel,
        mesh=mesh,
        out_type=jax.ShapeDtypeStruct((B,), jnp.float32),
        compiler_params=pltpu.CompilerParams(needs_layout_passes=False),
        scratch_types=[
            pltpu.VMEM((T,), jnp.float32),
            pltpu.VMEM((b_per_w,), jnp.int32),
            pltpu.VMEM((b_per_w,), jnp.float32),
            pltpu.SemaphoreType.DMA,
            pltpu.SemaphoreType.DMA,
        ],
    )
    def k(table_hbm, idx_hbm, out_hbm, table_v, idx_v, out_v, sem_t, sem_i):
        wid = lax.axis_index("s")
        base = wid * b_per_w
        cp_t = pltpu.async_copy(table_hbm, table_v, sem_t)
        cp_i = pltpu.async_copy(
            idx_hbm.at[pl.ds(base, b_per_w)], idx_v, sem_i)
        cp_i.wait()
        cp_t.wait()

        @plsc.parallel_loop(0, b_per_w, step=_LANES, unroll=8)
        def _gather(i):
            ids = idx_v[pl.ds(i, _LANES)]
            out_v[pl.ds(i, _LANES)] = plsc.load_gather(table_v, [ids])

        pltpu.sync_copy(out_v, out_hbm.at[pl.ds(base, b_per_w)])

    return k(table, idx)


def kernel(inData, inIndex, inShape):
    nbatch = inIndex.shape[0]
    out = _sc_gather(inData.astype(jnp.float32), inIndex.astype(jnp.int32))
    return out.reshape((nbatch,) + (1,) * (len(inShape) - 1))
